# Initial kernel scaffold; baseline (speedup 1.0000x reference)
import jax
import jax.numpy as jnp
from jax.experimental import pallas as pl


def kernel(x, edge_index, e, xbatch, bn_node_g, bn_node_b, bn_edge_g, bn_edge_b, lin0_W, lin0_b, attn_beta, lin_W, lin_b, em_W1, em_b1, em_W2, em_b2, em_W3, em_b3, em_W4, em_b4, em_W5, em_b5):
    raise NotImplementedError("write your pallas kernel here")


# speedup vs baseline: 1.2590x; 1.2590x over previous
import jax
import jax.numpy as jnp
from jax import lax
from jax.experimental import pallas as pl
from jax.experimental.pallas import tpu as pltpu
from jax.experimental.pallas import tpu_sc as plsc

N = 10000
D = 128
H = 256
HH = 128
E_EDGES = 320000
DE = 16
EPS = 1e-5
LEAK = 0.1
NUM_MP = 3

NC = 2
NS = 16
NW = NC * NS

K1 = 128
K2 = 256
K3 = 256
EL = E_EDGES + N
EP = ((EL + NW * K2 - 1) // (NW * K2)) * (NW * K2)
C1 = EP // (NW * K1)
C2 = EP // (NS * K2)
C3 = 40
EPF = NW * K3 * C3

NB = 10
EBLK = 20
BE = 3200


def _colstats_body(x_ref, o_ref):
    @pl.when(pl.program_id(0) == 0)
    def _():
        o_ref[...] = jnp.zeros_like(o_ref)

    blk = x_ref[...]
    s1 = jnp.sum(blk, axis=0, keepdims=True)
    s2 = jnp.sum(blk * blk, axis=0, keepdims=True)
    o_ref[...] += jnp.concatenate([s1, s2], axis=0)


def _colstats(x, nblocks):
    rows, feats = x.shape
    rb = rows // nblocks
    return pl.pallas_call(
        _colstats_body,
        grid=(nblocks,),
        in_specs=[pl.BlockSpec((rb, feats), lambda i: (i, 0))],
        out_specs=pl.BlockSpec((2, feats), lambda i: (0, 0)),
        out_shape=jax.ShapeDtypeStruct((2, feats), jnp.float32),
    )(x)


def _node0_body(x_ref, st_ref, g_ref, b_ref, W_ref, wb_ref,
                xlo_ref, xhi_ref, xnlo_ref, xnhi_ref):
    st = st_ref[...]
    m = st[0:1, :] / N
    v = st[1:2, :] / N - m * m
    xb = (x_ref[...] - m) / jnp.sqrt(v + EPS) * g_ref[...] + b_ref[...]
    y = jnp.dot(xb, W_ref[...]) + wb_ref[...]
    rn = 1.0 / jnp.clip(jnp.sqrt(jnp.sum(y * y, axis=1, keepdims=True)),
                        1e-12, None)
    yn = y * rn
    xlo_ref[...] = y[:, :HH]
    xhi_ref[...] = y[:, HH:]
    xnlo_ref[...] = yn[:, :HH]
    xnhi_ref[...] = yn[:, HH:]


def _node0(x, xstats, g, b, W, wb):
    rb = N // NB
    blk = pl.BlockSpec((rb, HH), lambda i: (i, 0))
    return pl.pallas_call(
        _node0_body,
        grid=(NB,),
        in_specs=[
            pl.BlockSpec((rb, D), lambda i: (i, 0)),
            pl.BlockSpec((2, D), lambda i: (0, 0)),
            pl.BlockSpec((1, D), lambda i: (0, 0)),
            pl.BlockSpec((1, D), lambda i: (0, 0)),
            pl.BlockSpec((D, H), lambda i: (0, 0)),
            pl.BlockSpec((1, H), lambda i: (0, 0)),
        ],
        out_specs=[blk, blk, blk, blk],
        out_shape=[jax.ShapeDtypeStruct((N, HH), jnp.float32)] * 4,
    )(x, xstats, g, b, W, wb)


def _mp_mid_body(alo_ref, ahi_ref, dp_ref, W_ref, b_ref,
                 xlo_ref, xhi_ref, xnlo_ref, xnhi_ref):
    a = jnp.concatenate([alo_ref[...], ahi_ref[...]], axis=1)
    den = dp_ref[0] + dp_ref[1]
    xm = a / jnp.clip(den, 1e-16, None)
    y = jnp.dot(xm, W_ref[...]) + b_ref[...]
    rn = 1.0 / jnp.clip(jnp.sqrt(jnp.sum(y * y, axis=1, keepdims=True)),
                        1e-12, None)
    yn = y * rn
    xlo_ref[...] = y[:, :HH]
    xhi_ref[...] = y[:, HH:]
    xnlo_ref[...] = yn[:, :HH]
    xnhi_ref[...] = yn[:, HH:]


def _mp_mid(alo, ahi, dp3, W, b):
    rb = N // NB
    blk = pl.BlockSpec((rb, HH), lambda i: (i, 0))
    return pl.pallas_call(
        _mp_mid_body,
        grid=(NB,),
        in_specs=[
            blk,
            blk,
            pl.BlockSpec((NC, rb, 1), lambda i: (0, i, 0)),
            pl.BlockSpec((H, H), lambda i: (0, 0)),
            pl.BlockSpec((1, H), lambda i: (0, 0)),
        ],
        out_specs=[blk, blk, blk, blk],
        out_shape=[jax.ShapeDtypeStruct((N, HH), jnp.float32)] * 4,
    )(alo, ahi, dp3, W, b)


def _mp_last_body(alo_ref, ahi_ref, dp_ref, W_ref, b_ref, Wa_ref, Wb_ref,
                  pp_ref):
    a = jnp.concatenate([alo_ref[...], ahi_ref[...]], axis=1)
    den = dp_ref[0] + dp_ref[1]
    xm = a / jnp.clip(den, 1e-16, None)
    y = jnp.dot(xm, W_ref[...]) + b_ref[...]
    pp_ref[...] = jnp.concatenate(
        [jnp.dot(y, Wa_ref[...]), jnp.dot(y, Wb_ref[...])], axis=1)


def _mp_last(alo, ahi, dp3, W, b, W1a, W1b):
    rb = N // NB
    blk = pl.BlockSpec((rb, HH), lambda i: (i, 0))
    oblk = pl.BlockSpec((rb, 128), lambda i: (i, 0))
    return pl.pallas_call(
        _mp_last_body,
        grid=(NB,),
        in_specs=[
            blk,
            blk,
            pl.BlockSpec((NC, rb, 1), lambda i: (0, i, 0)),
            pl.BlockSpec((H, H), lambda i: (0, 0)),
            pl.BlockSpec((1, H), lambda i: (0, 0)),
            pl.BlockSpec((H, 64), lambda i: (0, 0)),
            pl.BlockSpec((H, 64), lambda i: (0, 0)),
        ],
        out_specs=oblk,
        out_shape=jax.ShapeDtypeStruct((N, 128), jnp.float32),
    )(alo, ahi, dp3, W, b, W1a, W1b)


def _mlp_body(G_ref, e_ref, est_ref, eg_ref, ebb_ref, W1e_ref, b1_ref,
              W2_ref, b2_ref, W3_ref, b3_ref, W4_ref, b4_ref, W5_ref, b5_ref,
              o_ref):
    st = est_ref[...]
    m = st[0:1, :] / E_EDGES
    v = st[1:2, :] / E_EDGES - m * m
    sc = eg_ref[...] / jnp.sqrt(v + EPS)
    ebn = (e_ref[...] - m) * sc + ebb_ref[...]
    h = G_ref[...][:, :64] + jnp.dot(ebn, W1e_ref[...]) + b1_ref[...]
    h = jnp.where(h >= 0, h, LEAK * h)
    h = jnp.dot(h, W2_ref[...]) + b2_ref[...]
    h = jnp.where(h >= 0, h, LEAK * h)
    h = jnp.dot(h, W3_ref[...]) + b3_ref[...]
    h = jnp.where(h >= 0, h, LEAK * h)
    h = jnp.dot(h, W4_ref[...]) + b4_ref[...]
    h = jnp.where(h >= 0, h, LEAK * h)
    o_ref[...] = jnp.dot(h, W5_ref[...]) + b5_ref[...]


def _mlp(G, e, estats, eg, ebb, W1e, b1, W2, b2, W3, b3, W4, b4, W5, b5):
    nblk = E_EDGES // BE

    def _c(shape):
        return pl.BlockSpec(shape, lambda i: tuple(0 for _ in shape))

    return pl.pallas_call(
        _mlp_body,
        grid=(nblk,),
        in_specs=[
            pl.BlockSpec((BE, 128), lambda i: (i, 0)),
            pl.BlockSpec((BE, DE), lambda i: (i, 0)),
            _c((2, DE)), _c((1, DE)), _c((1, DE)),
            _c((DE, 64)), _c((1, 64)),
            _c((64, 32)), _c((1, 32)),
            _c((32, 16)), _c((1, 16)),
            _c((16, 8)), _c((1, 8)),
            _c((8, 2)), _c((1, 2)),
        ],
        out_specs=pl.BlockSpec((BE, 2), lambda i: (i, 0)),
        out_shape=jax.ShapeDtypeStruct((E_EDGES, 2), jnp.float32),
    )(G, e, estats, eg, ebb, W1e, b1, W2, b2, W3, b3, W4, b4, W5, b5)


_MESH = dict(core_axis_name="c", subcore_axis_name="s")
_SC_PARAMS = pltpu.CompilerParams(needs_layout_passes=False)


def _sc_pass1_body(xnlo, xnhi, srcp, dstp, bsv, zn,
                   ex_out, dparts,
                   sidx, didx, xslo, xshi, xdlo, xdhi, exbuf, bsv_v, den_sh,
                   den_vm, sem):
    c = lax.axis_index("c")
    s = lax.axis_index("s")
    wid = c * NS + s
    pltpu.sync_copy(bsv, bsv_v)

    @pl.when(s == 0)
    def _():
        pltpu.sync_copy(zn, den_sh)

    plsc.subcore_barrier()
    iota16 = lax.iota(jnp.int32, 16)
    rows_list = [g * 16 + iota16 for g in range(K1 // 16)]

    def chunk(k, carry):
        base = (wid * C1 + k) * K1
        pltpu.sync_copy(srcp.at[pl.ds(base, K1)], sidx)
        pltpu.sync_copy(dstp.at[pl.ds(base, K1)], didx)
        cps = [pltpu.async_copy(xnlo.at[sidx], xslo, sem),
               pltpu.async_copy(xnhi.at[sidx], xshi, sem),
               pltpu.async_copy(xnlo.at[didx], xdlo, sem),
               pltpu.async_copy(xnhi.at[didx], xdhi, sem)]
        for cp in cps:
            cp.wait()
        bv = bsv_v[0]
        sv = bsv_v[1]

        def fstep(f, accs):
            colf = jnp.full((16,), f, dtype=jnp.int32)
            out = []
            for g in range(K1 // 16):
                rg = rows_list[g]
                a1 = plsc.load_gather(xslo, [rg, colf])
                b1 = plsc.load_gather(xdlo, [rg, colf])
                a2 = plsc.load_gather(xshi, [rg, colf])
                b2 = plsc.load_gather(xdhi, [rg, colf])
                out.append(accs[g] + a1 * b1 + a2 * b2)
            return tuple(out)

        accs = lax.fori_loop(0, HH, fstep,
                             tuple(jnp.zeros((16,), jnp.float32)
                                   for _ in range(K1 // 16)))
        for g in range(K1 // 16):
            exv = jnp.exp(bv * accs[g] - sv)
            ge = base + g * 16 + iota16
            exv = jnp.where(ge < EL, exv, 0.0)
            exbuf[pl.ds(g * 16, 16)] = exv
        pltpu.sync_copy(exbuf, ex_out.at[pl.ds(base, K1)])
        pltpu.sync_copy(exbuf, den_sh.at[didx], add=True)
        return carry

    lax.fori_loop(0, C1, chunk, 0)
    plsc.subcore_barrier()

    @pl.when(s == 0)
    def _():
        pltpu.sync_copy(den_sh, den_vm)
        pltpu.sync_copy(den_vm, dparts.at[pl.ds(c * N, N)])


def _sc_pass1(xnlo, xnhi, srcp, dstp, bsv, zn):
    fn = pl.kernel(
        _sc_pass1_body,
        out_type=(jax.ShapeDtypeStruct((EP,), jnp.float32),
                  jax.ShapeDtypeStruct((NC * N,), jnp.float32)),
        mesh=plsc.VectorSubcoreMesh(**_MESH),
        compiler_params=_SC_PARAMS,
        scratch_types=[
            pltpu.VMEM((K1,), jnp.int32),
            pltpu.VMEM((K1,), jnp.int32),
            pltpu.VMEM((K1, HH), jnp.float32),
            pltpu.VMEM((K1, HH), jnp.float32),
            pltpu.VMEM((K1, HH), jnp.float32),
            pltpu.VMEM((K1, HH), jnp.float32),
            pltpu.VMEM((K1,), jnp.float32),
            pltpu.VMEM((2, 16), jnp.float32),
            pltpu.VMEM_SHARED((N,), jnp.float32),
            pltpu.VMEM((N,), jnp.float32),
            pltpu.SemaphoreType.DMA,
        ],
    )
    return fn(xnlo, xnhi, srcp, dstp, bsv, zn)


def _sc_pass2_body(xlo, xhi, srcp, dstp, ex_hbm, zacc,
                   acc_lo, acc_hi,
                   sidx, didx, exv_v, rows_v, acc_sh, sem):
    c = lax.axis_index("c")
    s = lax.axis_index("s")

    @pl.when(s == 0)
    def _():
        pltpu.sync_copy(zacc, acc_sh)

    plsc.subcore_barrier()
    iota16 = lax.iota(jnp.int32, 16)
    rows_list = [g * 16 + iota16 for g in range(K2 // 16)]

    def chunk(k, carry):
        base = (s * C2 + k) * K2
        pltpu.sync_copy(srcp.at[pl.ds(base, K2)], sidx)
        pltpu.sync_copy(dstp.at[pl.ds(base, K2)], didx)
        pltpu.sync_copy(ex_hbm.at[pl.ds(base, K2)], exv_v)

        @pl.when(c == 0)
        def _():
            pltpu.async_copy(xlo.at[sidx], rows_v, sem).wait()

        @pl.when(c == 1)
        def _():
            pltpu.async_copy(xhi.at[sidx], rows_v, sem).wait()

        exgs = [exv_v[pl.ds(g * 16, 16)] for g in range(K2 // 16)]

        def fstep(f, carry2):
            colf = jnp.full((16,), f, dtype=jnp.int32)
            for g in range(K2 // 16):
                rg = rows_list[g]
                val = plsc.load_gather(rows_v, [rg, colf])
                plsc.store_scatter(rows_v, [rg, colf], val * exgs[g])
            return carry2

        lax.fori_loop(0, HH, fstep, 0)
        pltpu.sync_copy(rows_v, acc_sh.at[didx], add=True)
        return carry

    lax.fori_loop(0, C2, chunk, 0)
    plsc.subcore_barrier()

    @pl.when(jnp.logical_and(s == 0, c == 0))
    def _():
        pltpu.sync_copy(acc_sh, acc_lo)

    @pl.when(jnp.logical_and(s == 0, c == 1))
    def _():
        pltpu.sync_copy(acc_sh, acc_hi)


def _sc_pass2(xlo, xhi, srcp, dstp, ex, zacc):
    fn = pl.kernel(
        _sc_pass2_body,
        out_type=(jax.ShapeDtypeStruct((N, HH), jnp.float32),
                  jax.ShapeDtypeStruct((N, HH), jnp.float32)),
        mesh=plsc.VectorSubcoreMesh(**_MESH),
        compiler_params=_SC_PARAMS,
        scratch_types=[
            pltpu.VMEM((K2,), jnp.int32),
            pltpu.VMEM((K2,), jnp.int32),
            pltpu.VMEM((K2,), jnp.float32),
            pltpu.VMEM((K2, HH), jnp.float32),
            pltpu.VMEM_SHARED((N, HH), jnp.float32),
            pltpu.SemaphoreType.DMA,
        ],
    )
    return fn(xlo, xhi, srcp, dstp, ex, zacc)


def _sc_gfin_body(pp, srcp, dstp, g_out, sidx, didx, gs_v, gd_v, sem):
    c = lax.axis_index("c")
    s = lax.axis_index("s")
    wid = c * NS + s
    iota16 = lax.iota(jnp.int32, 16)
    rows_list = [g * 16 + iota16 for g in range(K3 // 16)]

    def chunk(k, carry):
        base = (wid * C3 + k) * K3
        pltpu.sync_copy(srcp.at[pl.ds(base, K3)], sidx)
        pltpu.sync_copy(dstp.at[pl.ds(base, K3)], didx)
        cp1 = pltpu.async_copy(pp.at[sidx], gs_v, sem)
        cp2 = pltpu.async_copy(pp.at[didx], gd_v, sem)
        cp1.wait()
        cp2.wait()

        def fstep(f, carry2):
            colf = jnp.full((16,), f, dtype=jnp.int32)
            colf2 = colf + 64
            for g in range(K3 // 16):
                rg = rows_list[g]
                a = plsc.load_gather(gs_v, [rg, colf])
                b = plsc.load_gather(gd_v, [rg, colf2])
                plsc.store_scatter(gs_v, [rg, colf], a + b)
            return carry2

        lax.fori_loop(0, 64, fstep, 0)
        pltpu.sync_copy(gs_v, g_out.at[pl.ds(base, K3)])
        return carry

    lax.fori_loop(0, C3, chunk, 0)


def _sc_gfin(pp, srcp, dstp):
    fn = pl.kernel(
        _sc_gfin_body,
        out_type=jax.ShapeDtypeStruct((EPF, 128), jnp.float32),
        mesh=plsc.VectorSubcoreMesh(**_MESH),
        compiler_params=_SC_PARAMS,
        scratch_types=[
            pltpu.VMEM((K3,), jnp.int32),
            pltpu.VMEM((K3,), jnp.int32),
            pltpu.VMEM((K3, 128), jnp.float32),
            pltpu.VMEM((K3, 128), jnp.float32),
            pltpu.SemaphoreType.DMA,
        ],
    )
    return fn(pp, srcp, dstp)


def kernel(x, edge_index, e, xbatch, bn_node_g, bn_node_b, bn_edge_g,
           bn_edge_b, lin0_W, lin0_b, attn_beta, lin_W, lin_b,
           em_W1, em_b1, em_W2, em_b2, em_W3, em_b3, em_W4, em_b4,
           em_W5, em_b5):
    f32 = jnp.float32
    src = edge_index[0]
    dst = edge_index[1]
    loops = jnp.arange(N, dtype=jnp.int32)
    padi = jnp.arange(EP - EL, dtype=jnp.int32) % N
    srcp = jnp.concatenate([src, loops, padi])
    dstp = jnp.concatenate([dst, loops, padi])
    zn = jnp.zeros((N,), f32)
    zacc = jnp.zeros((N, HH), f32)

    xstats = _colstats(x, NB)
    estats = _colstats(e, EBLK)

    xlo, xhi, xnlo, xnhi = _node0(
        x, xstats, bn_node_g.reshape(1, -1), bn_node_b.reshape(1, -1),
        lin0_W, lin0_b.reshape(1, -1))

    pp = None
    for i in range(NUM_MP):
        bsv = jnp.stack([jnp.full((16,), attn_beta[i], dtype=f32),
                         jnp.full((16,), jnp.abs(attn_beta[i]), dtype=f32)])
        ex, dparts = _sc_pass1(xnlo, xnhi, srcp, dstp, bsv, zn)
        acc_lo, acc_hi = _sc_pass2(xlo, xhi, srcp, dstp, ex, zacc)
        dp3 = dparts.reshape(NC, N, 1)
        if i < NUM_MP - 1:
            xlo, xhi, xnlo, xnhi = _mp_mid(
                acc_lo, acc_hi, dp3, lin_W[i], lin_b[i].reshape(1, -1))
        else:
            pp = _mp_last(
                acc_lo, acc_hi, dp3, lin_W[i], lin_b[i].reshape(1, -1),
                em_W1[:H], em_W1[H:2 * H])

    G = _sc_gfin(pp, srcp, dstp)
    return _mlp(
        G, e, estats, bn_edge_g.reshape(1, -1), bn_edge_b.reshape(1, -1),
        em_W1[2 * H:], em_b1.reshape(1, -1), em_W2, em_b2.reshape(1, -1),
        em_W3, em_b3.reshape(1, -1), em_W4, em_b4.reshape(1, -1),
        em_W5, em_b5.reshape(1, -1))


# speedup vs baseline: 2.0411x; 1.6211x over previous
import jax
import jax.numpy as jnp
from jax import lax
from jax.experimental import pallas as pl
from jax.experimental.pallas import tpu as pltpu
from jax.experimental.pallas import tpu_sc as plsc

N = 10000
D = 128
H = 256
HH = 128
E_EDGES = 320000
DE = 16
EPS = 1e-5
LEAK = 0.1
NUM_MP = 3

NP2 = 10240
NSLAB = NP2 // 128

NC = 2
NS = 16
NW = NC * NS

K2 = 256
K3 = 256
EL = E_EDGES + N
EP = ((EL + NW * K2 - 1) // (NW * K2)) * (NW * K2)
C2 = EP // (NS * K2)
C3 = 40
EPF = NW * K3 * C3

NB = 10
GB = 1000
GBC = 1024
EBLK = 20
BE = 3200


def _colstats_body(x_ref, o_ref):
    @pl.when(pl.program_id(0) == 0)
    def _():
        o_ref[...] = jnp.zeros_like(o_ref)

    blk = x_ref[...]
    s1 = jnp.sum(blk, axis=0, keepdims=True)
    s2 = jnp.sum(blk * blk, axis=0, keepdims=True)
    o_ref[...] += jnp.concatenate([s1, s2], axis=0)


def _colstats(x, nblocks):
    rows, feats = x.shape
    rb = rows // nblocks
    return pl.pallas_call(
        _colstats_body,
        grid=(nblocks,),
        in_specs=[pl.BlockSpec((rb, feats), lambda i: (i, 0))],
        out_specs=pl.BlockSpec((2, feats), lambda i: (0, 0)),
        out_shape=jax.ShapeDtypeStruct((2, feats), jnp.float32),
    )(x)


def _node0_body(x_ref, st_ref, g_ref, b_ref, W_ref, wb_ref,
                xlo_ref, xhi_ref, xn_ref):
    st = st_ref[...]
    m = st[0:1, :] / N
    v = st[1:2, :] / N - m * m
    xb = (x_ref[...] - m) / jnp.sqrt(v + EPS) * g_ref[...] + b_ref[...]
    y = jnp.dot(xb, W_ref[...]) + wb_ref[...]
    rn = 1.0 / jnp.clip(jnp.sqrt(jnp.sum(y * y, axis=1, keepdims=True)),
                        1e-12, None)
    xlo_ref[...] = y[:, :HH]
    xhi_ref[...] = y[:, HH:]
    xn_ref[...] = y * rn


def _node0(x, xstats, g, b, W, wb):
    rb = N // NB
    blk = pl.BlockSpec((rb, HH), lambda i: (i, 0))
    fblk = pl.BlockSpec((rb, H), lambda i: (i, 0))
    return pl.pallas_call(
        _node0_body,
        grid=(NB,),
        in_specs=[
            pl.BlockSpec((rb, D), lambda i: (i, 0)),
            pl.BlockSpec((2, D), lambda i: (0, 0)),
            pl.BlockSpec((1, D), lambda i: (0, 0)),
            pl.BlockSpec((1, D), lambda i: (0, 0)),
            pl.BlockSpec((D, H), lambda i: (0, 0)),
            pl.BlockSpec((1, H), lambda i: (0, 0)),
        ],
        out_specs=[blk, blk, fblk],
        out_shape=[jax.ShapeDtypeStruct((N, HH), jnp.float32),
                   jax.ShapeDtypeStruct((N, HH), jnp.float32),
                   jax.ShapeDtypeStruct((N, H), jnp.float32)],
    )(x, xstats, g, b, W, wb)


def _gram_body(a_ref, b_ref, bs_ref, o_ref):
    g = lax.dot_general(a_ref[...], b_ref[...], (((1,), (1,)), ((), ())),
                        preferred_element_type=jnp.float32)
    ex = jnp.exp(bs_ref[0, 0] * g - bs_ref[1, 0])
    for t in range(GBC // 128):
        o_ref[:, t, :] = ex[:, t * 128:(t + 1) * 128]


def _gram(xnp, bs):
    return pl.pallas_call(
        _gram_body,
        grid=(NB, NP2 // GBC),
        in_specs=[
            pl.BlockSpec((GB, H), lambda i, j: (i, 0)),
            pl.BlockSpec((GBC, H), lambda i, j: (j, 0)),
            pl.BlockSpec((2, 1), lambda i, j: (0, 0)),
        ],
        out_specs=pl.BlockSpec((GB, GBC // 128, 128),
                               lambda i, j: (i, j, 0)),
        out_shape=jax.ShapeDtypeStruct((N, NSLAB, 128), jnp.float32),
    )(xnp, xnp, bs)


def _mp_mid_body(alo_ref, ahi_ref, dp_ref, W_ref, b_ref,
                 xlo_ref, xhi_ref, xn_ref):
    a = jnp.concatenate([alo_ref[...], ahi_ref[...]], axis=1)
    xm = a / jnp.clip(dp_ref[...], 1e-16, None)
    y = jnp.dot(xm, W_ref[...]) + b_ref[...]
    rn = 1.0 / jnp.clip(jnp.sqrt(jnp.sum(y * y, axis=1, keepdims=True)),
                        1e-12, None)
    xlo_ref[...] = y[:, :HH]
    xhi_ref[...] = y[:, HH:]
    xn_ref[...] = y * rn


def _mp_mid(alo, ahi, dp, W, b):
    rb = N // NB
    blk = pl.BlockSpec((rb, HH), lambda i: (i, 0))
    fblk = pl.BlockSpec((rb, H), lambda i: (i, 0))
    return pl.pallas_call(
        _mp_mid_body,
        grid=(NB,),
        in_specs=[
            blk,
            blk,
            pl.BlockSpec((rb, 1), lambda i: (i, 0)),
            pl.BlockSpec((H, H), lambda i: (0, 0)),
            pl.BlockSpec((1, H), lambda i: (0, 0)),
        ],
        out_specs=[blk, blk, fblk],
        out_shape=[jax.ShapeDtypeStruct((N, HH), jnp.float32),
                   jax.ShapeDtypeStruct((N, HH), jnp.float32),
                   jax.ShapeDtypeStruct((N, H), jnp.float32)],
    )(alo, ahi, dp, W, b)


def _mp_last_body(alo_ref, ahi_ref, dp_ref, W_ref, b_ref, Wa_ref, Wb_ref,
                  pp_ref):
    a = jnp.concatenate([alo_ref[...], ahi_ref[...]], axis=1)
    xm = a / jnp.clip(dp_ref[...], 1e-16, None)
    y = jnp.dot(xm, W_ref[...]) + b_ref[...]
    pp_ref[...] = jnp.concatenate(
        [jnp.dot(y, Wa_ref[...]), jnp.dot(y, Wb_ref[...])], axis=1)


def _mp_last(alo, ahi, dp, W, b, W1a, W1b):
    rb = N // NB
    blk = pl.BlockSpec((rb, HH), lambda i: (i, 0))
    oblk = pl.BlockSpec((rb, 128), lambda i: (i, 0))
    return pl.pallas_call(
        _mp_last_body,
        grid=(NB,),
        in_specs=[
            blk,
            blk,
            pl.BlockSpec((rb, 1), lambda i: (i, 0)),
            pl.BlockSpec((H, H), lambda i: (0, 0)),
            pl.BlockSpec((1, H), lambda i: (0, 0)),
            pl.BlockSpec((H, 64), lambda i: (0, 0)),
            pl.BlockSpec((H, 64), lambda i: (0, 0)),
        ],
        out_specs=oblk,
        out_shape=jax.ShapeDtypeStruct((N, 128), jnp.float32),
    )(alo, ahi, dp, W, b, W1a, W1b)


def _mlp_body(G_ref, e_ref, est_ref, eg_ref, ebb_ref, W1e_ref, b1_ref,
              W2_ref, b2_ref, W3_ref, b3_ref, W4_ref, b4_ref, W5_ref, b5_ref,
              o_ref):
    st = est_ref[...]
    m = st[0:1, :] / E_EDGES
    v = st[1:2, :] / E_EDGES - m * m
    sc = eg_ref[...] / jnp.sqrt(v + EPS)
    ebn = (e_ref[...] - m) * sc + ebb_ref[...]
    h = G_ref[...][:, :64] + jnp.dot(ebn, W1e_ref[...]) + b1_ref[...]
    h = jnp.where(h >= 0, h, LEAK * h)
    h = jnp.dot(h, W2_ref[...]) + b2_ref[...]
    h = jnp.where(h >= 0, h, LEAK * h)
    h = jnp.dot(h, W3_ref[...]) + b3_ref[...]
    h = jnp.where(h >= 0, h, LEAK * h)
    h = jnp.dot(h, W4_ref[...]) + b4_ref[...]
    h = jnp.where(h >= 0, h, LEAK * h)
    o_ref[...] = jnp.dot(h, W5_ref[...]) + b5_ref[...]


def _mlp(G, e, estats, eg, ebb, W1e, b1, W2, b2, W3, b3, W4, b4, W5, b5):
    nblk = E_EDGES // BE

    def _c(shape):
        return pl.BlockSpec(shape, lambda i: tuple(0 for _ in shape))

    return pl.pallas_call(
        _mlp_body,
        grid=(nblk,),
        in_specs=[
            pl.BlockSpec((BE, 128), lambda i: (i, 0)),
            pl.BlockSpec((BE, DE), lambda i: (i, 0)),
            _c((2, DE)), _c((1, DE)), _c((1, DE)),
            _c((DE, 64)), _c((1, 64)),
            _c((64, 32)), _c((1, 32)),
            _c((32, 16)), _c((1, 16)),
            _c((16, 8)), _c((1, 8)),
            _c((8, 2)), _c((1, 2)),
        ],
        out_specs=pl.BlockSpec((BE, 2), lambda i: (i, 0)),
        out_shape=jax.ShapeDtypeStruct((E_EDGES, 2), jnp.float32),
    )(G, e, estats, eg, ebb, W1e, b1, W2, b2, W3, b3, W4, b4, W5, b5)


_MESH = dict(core_axis_name="c", subcore_axis_name="s")
_SC_PARAMS = pltpu.CompilerParams(needs_layout_passes=False)


def _sc_round_body(xlo, xhi, srcp, dstp, ext, zacc, zn,
                   acc_lo, acc_hi, den_out,
                   sidx, didx, ridx, exbuf, rows_v,
                   acc_sh, den_sh, sem):
    c = lax.axis_index("c")
    s = lax.axis_index("s")

    @pl.when(s == 0)
    def _():
        pltpu.sync_copy(zacc, acc_sh)

    @pl.when(jnp.logical_and(s == 0, c == 0))
    def _():
        pltpu.sync_copy(zn, den_sh)

    plsc.subcore_barrier()
    iota16 = lax.iota(jnp.int32, 16)
    rows_list = [g * 16 + iota16 for g in range(K2 // 16)]

    def chunk(k, carry):
        base = (s * C2 + k) * K2
        pltpu.sync_copy(srcp.at[pl.ds(base, K2)], sidx)
        pltpu.sync_copy(dstp.at[pl.ds(base, K2)], didx)
        for g in range(K2 // 16):
            sv = sidx[pl.ds(g * 16, 16)]
            dv = didx[pl.ds(g * 16, 16)]
            ridx[pl.ds(g * 16, 16)] = (
                dv * NSLAB + lax.shift_right_logical(sv, 7))
        pltpu.async_copy(ext.at[ridx], rows_v, sem).wait()
        for g in range(K2 // 16):
            sv = sidx[pl.ds(g * 16, 16)]
            colv = lax.bitwise_and(sv, 127)
            exv = plsc.load_gather(rows_v, [rows_list[g], colv])
            ge = base + g * 16 + iota16
            exbuf[pl.ds(g * 16, 16)] = jnp.where(ge < EL, exv, 0.0)

        @pl.when(c == 0)
        def _():
            pltpu.sync_copy(exbuf, den_sh.at[didx], add=True)

        @pl.when(c == 0)
        def _():
            pltpu.async_copy(xlo.at[sidx], rows_v, sem).wait()

        @pl.when(c == 1)
        def _():
            pltpu.async_copy(xhi.at[sidx], rows_v, sem).wait()

        exgs = [exbuf[pl.ds(g * 16, 16)] for g in range(K2 // 16)]

        def fstep(f, carry2):
            colf = jnp.full((16,), f, dtype=jnp.int32)
            for g in range(K2 // 16):
                rg = rows_list[g]
                val = plsc.load_gather(rows_v, [rg, colf])
                plsc.store_scatter(rows_v, [rg, colf], val * exgs[g])
            return carry2

        lax.fori_loop(0, HH, fstep, 0)
        pltpu.sync_copy(rows_v, acc_sh.at[didx], add=True)
        return carry

    lax.fori_loop(0, C2, chunk, 0)
    plsc.subcore_barrier()

    @pl.when(jnp.logical_and(s == 0, c == 0))
    def _():
        pltpu.sync_copy(acc_sh, acc_lo)
        pltpu.sync_copy(den_sh, den_out)

    @pl.when(jnp.logical_and(s == 0, c == 1))
    def _():
        pltpu.sync_copy(acc_sh, acc_hi)


def _sc_round(xlo, xhi, srcp, dstp, ext, zacc, zn):
    fn = pl.kernel(
        _sc_round_body,
        out_type=(jax.ShapeDtypeStruct((N, HH), jnp.float32),
                  jax.ShapeDtypeStruct((N, HH), jnp.float32),
                  jax.ShapeDtypeStruct((N,), jnp.float32)),
        mesh=plsc.VectorSubcoreMesh(**_MESH),
        compiler_params=_SC_PARAMS,
        scratch_types=[
            pltpu.VMEM((K2,), jnp.int32),
            pltpu.VMEM((K2,), jnp.int32),
            pltpu.VMEM((K2,), jnp.int32),
            pltpu.VMEM((K2,), jnp.float32),
            pltpu.VMEM((K2, HH), jnp.float32),
            pltpu.VMEM_SHARED((N, HH), jnp.float32),
            pltpu.VMEM_SHARED((N,), jnp.float32),
            pltpu.SemaphoreType.DMA,
        ],
    )
    return fn(xlo, xhi, srcp, dstp, ext, zacc, zn)


def _sc_gfin_body(pp, srcp, dstp, g_out, sidx, didx, gs_v, gd_v, sem):
    c = lax.axis_index("c")
    s = lax.axis_index("s")
    wid = c * NS + s
    iota16 = lax.iota(jnp.int32, 16)
    rows_list = [g * 16 + iota16 for g in range(K3 // 16)]

    def chunk(k, carry):
        base = (wid * C3 + k) * K3
        pltpu.sync_copy(srcp.at[pl.ds(base, K3)], sidx)
        pltpu.sync_copy(dstp.at[pl.ds(base, K3)], didx)
        cp1 = pltpu.async_copy(pp.at[sidx], gs_v, sem)
        cp2 = pltpu.async_copy(pp.at[didx], gd_v, sem)
        cp1.wait()
        cp2.wait()

        def fstep(f, carry2):
            colf = jnp.full((16,), f, dtype=jnp.int32)
            colf2 = colf + 64
            for g in range(K3 // 16):
                rg = rows_list[g]
                a = plsc.load_gather(gs_v, [rg, colf])
                b = plsc.load_gather(gd_v, [rg, colf2])
                plsc.store_scatter(gs_v, [rg, colf], a + b)
            return carry2

        lax.fori_loop(0, 64, fstep, 0)
        pltpu.sync_copy(gs_v, g_out.at[pl.ds(base, K3)])
        return carry

    lax.fori_loop(0, C3, chunk, 0)


def _sc_gfin(pp, srcp, dstp):
    fn = pl.kernel(
        _sc_gfin_body,
        out_type=jax.ShapeDtypeStruct((EPF, 128), jnp.float32),
        mesh=plsc.VectorSubcoreMesh(**_MESH),
        compiler_params=_SC_PARAMS,
        scratch_types=[
            pltpu.VMEM((K3,), jnp.int32),
            pltpu.VMEM((K3,), jnp.int32),
            pltpu.VMEM((K3, 128), jnp.float32),
            pltpu.VMEM((K3, 128), jnp.float32),
            pltpu.SemaphoreType.DMA,
        ],
    )
    return fn(pp, srcp, dstp)


def kernel(x, edge_index, e, xbatch, bn_node_g, bn_node_b, bn_edge_g,
           bn_edge_b, lin0_W, lin0_b, attn_beta, lin_W, lin_b,
           em_W1, em_b1, em_W2, em_b2, em_W3, em_b3, em_W4, em_b4,
           em_W5, em_b5):
    f32 = jnp.float32
    src = edge_index[0]
    dst = edge_index[1]
    loops = jnp.arange(N, dtype=jnp.int32)
    padi = jnp.arange(EP - EL, dtype=jnp.int32) % N
    srcp = jnp.concatenate([src, loops, padi])
    dstp = jnp.concatenate([dst, loops, padi])
    zn = jnp.zeros((N,), f32)
    zacc = jnp.zeros((N, HH), f32)

    xstats = _colstats(x, NB)
    estats = _colstats(e, EBLK)

    xlo, xhi, xn = _node0(
        x, xstats, bn_node_g.reshape(1, -1), bn_node_b.reshape(1, -1),
        lin0_W, lin0_b.reshape(1, -1))

    pp = None
    for i in range(NUM_MP):
        bs = jnp.stack([attn_beta[i], jnp.abs(attn_beta[i])]).reshape(2, 1)
        xnp = jnp.pad(xn, ((0, NP2 - N), (0, 0)))
        ext = _gram(xnp, bs).reshape(N * NSLAB, 128)
        acc_lo, acc_hi, den = _sc_round(xlo, xhi, srcp, dstp, ext, zacc, zn)
        dp = den.reshape(N, 1)
        if i < NUM_MP - 1:
            xlo, xhi, xn = _mp_mid(
                acc_lo, acc_hi, dp, lin_W[i], lin_b[i].reshape(1, -1))
        else:
            pp = _mp_last(
                acc_lo, acc_hi, dp, lin_W[i], lin_b[i].reshape(1, -1),
                em_W1[:H], em_W1[H:2 * H])

    G = _sc_gfin(pp, srcp, dstp)
    return _mlp(
        G, e, estats, bn_edge_g.reshape(1, -1), bn_edge_b.reshape(1, -1),
        em_W1[2 * H:], em_b1.reshape(1, -1), em_W2, em_b2.reshape(1, -1),
        em_W3, em_b3.reshape(1, -1), em_W4, em_b4.reshape(1, -1),
        em_W5, em_b5.reshape(1, -1))


# speedup vs baseline: 4.6182x; 2.2626x over previous
import jax
import jax.numpy as jnp
from jax import lax
from jax.experimental import pallas as pl
from jax.experimental.pallas import tpu as pltpu
from jax.experimental.pallas import tpu_sc as plsc

N = 10000
D = 128
H = 256
HH = 128
E_EDGES = 320000
DE = 16
EPS = 1e-5
LEAK = 0.1
NUM_MP = 3

NP2 = 10240
NSLAB = NP2 // 128

NC = 2
NS = 16
NW = NC * NS

K2 = 128
K3 = 256
EL = E_EDGES + N
EP = ((EL + NW * K2 - 1) // (NW * K2)) * (NW * K2)
C2 = EP // (NS * K2)
C3 = 40
EPF = NW * K3 * C3

NB = 10
GB = 1000
GBC = 1024
EBLK = 20
BE = 3200


def _colstats_body(x_ref, o_ref):
    @pl.when(pl.program_id(0) == 0)
    def _():
        o_ref[...] = jnp.zeros_like(o_ref)

    blk = x_ref[...]
    s1 = jnp.sum(blk, axis=0, keepdims=True)
    s2 = jnp.sum(blk * blk, axis=0, keepdims=True)
    o_ref[...] += jnp.concatenate([s1, s2], axis=0)


def _colstats(x, nblocks):
    rows, feats = x.shape
    rb = rows // nblocks
    return pl.pallas_call(
        _colstats_body,
        grid=(nblocks,),
        in_specs=[pl.BlockSpec((rb, feats), lambda i: (i, 0))],
        out_specs=pl.BlockSpec((2, feats), lambda i: (0, 0)),
        out_shape=jax.ShapeDtypeStruct((2, feats), jnp.float32),
    )(x)


def _node0_body(x_ref, st_ref, g_ref, b_ref, W_ref, wb_ref,
                xlo_ref, xhi_ref, xn_ref):
    st = st_ref[...]
    m = st[0:1, :] / N
    v = st[1:2, :] / N - m * m
    xb = (x_ref[...] - m) / jnp.sqrt(v + EPS) * g_ref[...] + b_ref[...]
    y = jnp.dot(xb, W_ref[...]) + wb_ref[...]
    rn = 1.0 / jnp.clip(jnp.sqrt(jnp.sum(y * y, axis=1, keepdims=True)),
                        1e-12, None)
    xlo_ref[...] = y[:, :HH]
    xhi_ref[...] = y[:, HH:]
    xn_ref[...] = y * rn


def _node0(x, xstats, g, b, W, wb):
    rb = N // NB
    blk = pl.BlockSpec((rb, HH), lambda i: (i, 0))
    fblk = pl.BlockSpec((rb, H), lambda i: (i, 0))
    return pl.pallas_call(
        _node0_body,
        grid=(NB,),
        in_specs=[
            pl.BlockSpec((rb, D), lambda i: (i, 0)),
            pl.BlockSpec((2, D), lambda i: (0, 0)),
            pl.BlockSpec((1, D), lambda i: (0, 0)),
            pl.BlockSpec((1, D), lambda i: (0, 0)),
            pl.BlockSpec((D, H), lambda i: (0, 0)),
            pl.BlockSpec((1, H), lambda i: (0, 0)),
        ],
        out_specs=[blk, blk, fblk],
        out_shape=[jax.ShapeDtypeStruct((N, HH), jnp.float32),
                   jax.ShapeDtypeStruct((N, HH), jnp.float32),
                   jax.ShapeDtypeStruct((N, H), jnp.float32)],
    )(x, xstats, g, b, W, wb)


def _gram_body(a_ref, b_ref, bs_ref, o_ref):
    g = lax.dot_general(a_ref[...], b_ref[...], (((1,), (1,)), ((), ())),
                        preferred_element_type=jnp.float32)
    ex = jnp.exp(bs_ref[0, 0] * g - bs_ref[1, 0])
    for t in range(GBC // 128):
        o_ref[:, t, :] = ex[:, t * 128:(t + 1) * 128]


def _gram(xnp, bs):
    return pl.pallas_call(
        _gram_body,
        grid=(NB, NP2 // GBC),
        in_specs=[
            pl.BlockSpec((GB, H), lambda i, j: (i, 0)),
            pl.BlockSpec((GBC, H), lambda i, j: (j, 0)),
            pl.BlockSpec((2, 1), lambda i, j: (0, 0)),
        ],
        out_specs=pl.BlockSpec((GB, GBC // 128, 128),
                               lambda i, j: (i, j, 0)),
        out_shape=jax.ShapeDtypeStruct((N, NSLAB, 128), jnp.float32),
    )(xnp, xnp, bs)


def _mp_mid_body(alo_ref, ahi_ref, dp_ref, W_ref, b_ref,
                 xlo_ref, xhi_ref, xn_ref):
    a = jnp.concatenate([alo_ref[...], ahi_ref[...]], axis=1)
    xm = a / jnp.clip(dp_ref[...], 1e-16, None)
    y = jnp.dot(xm, W_ref[...]) + b_ref[...]
    rn = 1.0 / jnp.clip(jnp.sqrt(jnp.sum(y * y, axis=1, keepdims=True)),
                        1e-12, None)
    xlo_ref[...] = y[:, :HH]
    xhi_ref[...] = y[:, HH:]
    xn_ref[...] = y * rn


def _mp_mid(alo, ahi, dp, W, b):
    rb = N // NB
    blk = pl.BlockSpec((rb, HH), lambda i: (i, 0))
    fblk = pl.BlockSpec((rb, H), lambda i: (i, 0))
    return pl.pallas_call(
        _mp_mid_body,
        grid=(NB,),
        in_specs=[
            blk,
            blk,
            pl.BlockSpec((rb, 1), lambda i: (i, 0)),
            pl.BlockSpec((H, H), lambda i: (0, 0)),
            pl.BlockSpec((1, H), lambda i: (0, 0)),
        ],
        out_specs=[blk, blk, fblk],
        out_shape=[jax.ShapeDtypeStruct((N, HH), jnp.float32),
                   jax.ShapeDtypeStruct((N, HH), jnp.float32),
                   jax.ShapeDtypeStruct((N, H), jnp.float32)],
    )(alo, ahi, dp, W, b)


def _mp_last_body(alo_ref, ahi_ref, dp_ref, W_ref, b_ref, Wa_ref, Wb_ref,
                  pp_ref):
    a = jnp.concatenate([alo_ref[...], ahi_ref[...]], axis=1)
    xm = a / jnp.clip(dp_ref[...], 1e-16, None)
    y = jnp.dot(xm, W_ref[...]) + b_ref[...]
    pp_ref[...] = jnp.concatenate(
        [jnp.dot(y, Wa_ref[...]), jnp.dot(y, Wb_ref[...])], axis=1)


def _mp_last(alo, ahi, dp, W, b, W1a, W1b):
    rb = N // NB
    blk = pl.BlockSpec((rb, HH), lambda i: (i, 0))
    oblk = pl.BlockSpec((rb, 128), lambda i: (i, 0))
    return pl.pallas_call(
        _mp_last_body,
        grid=(NB,),
        in_specs=[
            blk,
            blk,
            pl.BlockSpec((rb, 1), lambda i: (i, 0)),
            pl.BlockSpec((H, H), lambda i: (0, 0)),
            pl.BlockSpec((1, H), lambda i: (0, 0)),
            pl.BlockSpec((H, 64), lambda i: (0, 0)),
            pl.BlockSpec((H, 64), lambda i: (0, 0)),
        ],
        out_specs=oblk,
        out_shape=jax.ShapeDtypeStruct((N, 128), jnp.float32),
    )(alo, ahi, dp, W, b, W1a, W1b)


def _mlp_body(G_ref, e_ref, est_ref, eg_ref, ebb_ref, W1e_ref, b1_ref,
              W2_ref, b2_ref, W3_ref, b3_ref, W4_ref, b4_ref, W5_ref, b5_ref,
              o_ref):
    st = est_ref[...]
    m = st[0:1, :] / E_EDGES
    v = st[1:2, :] / E_EDGES - m * m
    sc = eg_ref[...] / jnp.sqrt(v + EPS)
    ebn = (e_ref[...] - m) * sc + ebb_ref[...]
    h = G_ref[...][:, :64] + jnp.dot(ebn, W1e_ref[...]) + b1_ref[...]
    h = jnp.where(h >= 0, h, LEAK * h)
    h = jnp.dot(h, W2_ref[...]) + b2_ref[...]
    h = jnp.where(h >= 0, h, LEAK * h)
    h = jnp.dot(h, W3_ref[...]) + b3_ref[...]
    h = jnp.where(h >= 0, h, LEAK * h)
    h = jnp.dot(h, W4_ref[...]) + b4_ref[...]
    h = jnp.where(h >= 0, h, LEAK * h)
    o_ref[...] = jnp.dot(h, W5_ref[...]) + b5_ref[...]


def _mlp(G, e, estats, eg, ebb, W1e, b1, W2, b2, W3, b3, W4, b4, W5, b5):
    nblk = E_EDGES // BE

    def _c(shape):
        return pl.BlockSpec(shape, lambda i: tuple(0 for _ in shape))

    return pl.pallas_call(
        _mlp_body,
        grid=(nblk,),
        in_specs=[
            pl.BlockSpec((BE, 128), lambda i: (i, 0)),
            pl.BlockSpec((BE, DE), lambda i: (i, 0)),
            _c((2, DE)), _c((1, DE)), _c((1, DE)),
            _c((DE, 64)), _c((1, 64)),
            _c((64, 32)), _c((1, 32)),
            _c((32, 16)), _c((1, 16)),
            _c((16, 8)), _c((1, 8)),
            _c((8, 2)), _c((1, 2)),
        ],
        out_specs=pl.BlockSpec((BE, 2), lambda i: (i, 0)),
        out_shape=jax.ShapeDtypeStruct((E_EDGES, 2), jnp.float32),
    )(G, e, estats, eg, ebb, W1e, b1, W2, b2, W3, b3, W4, b4, W5, b5)


_MESH = dict(core_axis_name="c", subcore_axis_name="s")
_SC_PARAMS = pltpu.CompilerParams(needs_layout_passes=False)


def _sc_round_body(xlo, xhi, srcp, dstp, ext, zacc, zn,
                   acc_lo, acc_hi, den_out,
                   sidx, didx, ridx, exbuf, rows_v, sc_v,
                   acc_sh, den_sh, sem):
    c = lax.axis_index("c")
    s = lax.axis_index("s")

    @pl.when(s == 0)
    def _():
        pltpu.sync_copy(zacc, acc_sh)

    @pl.when(jnp.logical_and(s == 0, c == 0))
    def _():
        pltpu.sync_copy(zn, den_sh)

    plsc.subcore_barrier()
    iota16 = lax.iota(jnp.int32, 16)
    rows_list = [g * 16 + iota16 for g in range(K2 // 16)]

    def chunk(k, carry):
        base = (s * C2 + k) * K2
        pltpu.sync_copy(srcp.at[pl.ds(base, K2)], sidx)
        pltpu.sync_copy(dstp.at[pl.ds(base, K2)], didx)
        for g in range(K2 // 16):
            sv = sidx[pl.ds(g * 16, 16)]
            dv = didx[pl.ds(g * 16, 16)]
            ridx[pl.ds(g * 16, 16)] = (
                dv * NSLAB + lax.shift_right_logical(sv, 7))
        pltpu.async_copy(ext.at[ridx], rows_v, sem).wait()
        for g in range(K2 // 16):
            sv = sidx[pl.ds(g * 16, 16)]
            colv = lax.bitwise_and(sv, 127)
            exv = plsc.load_gather(rows_v, [rows_list[g], colv])
            ge = base + g * 16 + iota16
            exbuf[pl.ds(g * 16, 16)] = jnp.where(ge < EL, exv, 0.0)

        @pl.when(c == 0)
        def _():
            pltpu.sync_copy(exbuf, den_sh.at[didx], add=True)

        @pl.when(c == 0)
        def _():
            pltpu.async_copy(xlo.at[sidx], rows_v, sem).wait()

        @pl.when(c == 1)
        def _():
            pltpu.async_copy(xhi.at[sidx], rows_v, sem).wait()

        def rstep(r, carry2):
            vb = plsc.load_gather(exbuf, [jnp.full((16,), r, jnp.int32)])
            for t in range(HH // 16):
                sc_v[r, pl.ds(t * 16, 16)] = (
                    rows_v[r, pl.ds(t * 16, 16)] * vb)
            return carry2

        lax.fori_loop(0, K2, rstep, 0)
        pltpu.sync_copy(sc_v, acc_sh.at[didx], add=True)
        return carry

    lax.fori_loop(0, C2, chunk, 0)
    plsc.subcore_barrier()

    @pl.when(jnp.logical_and(s == 0, c == 0))
    def _():
        pltpu.sync_copy(acc_sh, acc_lo)
        pltpu.sync_copy(den_sh, den_out)

    @pl.when(jnp.logical_and(s == 0, c == 1))
    def _():
        pltpu.sync_copy(acc_sh, acc_hi)


def _sc_round(xlo, xhi, srcp, dstp, ext, zacc, zn):
    fn = pl.kernel(
        _sc_round_body,
        out_type=(jax.ShapeDtypeStruct((N, HH), jnp.float32),
                  jax.ShapeDtypeStruct((N, HH), jnp.float32),
                  jax.ShapeDtypeStruct((N,), jnp.float32)),
        mesh=plsc.VectorSubcoreMesh(**_MESH),
        compiler_params=_SC_PARAMS,
        scratch_types=[
            pltpu.VMEM((K2,), jnp.int32),
            pltpu.VMEM((K2,), jnp.int32),
            pltpu.VMEM((K2,), jnp.int32),
            pltpu.VMEM((K2,), jnp.float32),
            pltpu.VMEM((K2, HH), jnp.float32),
            pltpu.VMEM((K2, HH), jnp.float32),
            pltpu.VMEM_SHARED((N, HH), jnp.float32),
            pltpu.VMEM_SHARED((N,), jnp.float32),
            pltpu.SemaphoreType.DMA,
        ],
    )
    return fn(xlo, xhi, srcp, dstp, ext, zacc, zn)


def _sc_gfin_body(pp, srcp, dstp, g_out, sidx, didx, gs_v, gd_v, go_v, sem):
    c = lax.axis_index("c")
    s = lax.axis_index("s")
    wid = c * NS + s

    def chunk(k, carry):
        base = (wid * C3 + k) * K3
        pltpu.sync_copy(srcp.at[pl.ds(base, K3)], sidx)
        pltpu.sync_copy(dstp.at[pl.ds(base, K3)], didx)
        cp1 = pltpu.async_copy(pp.at[sidx], gs_v, sem)
        cp2 = pltpu.async_copy(pp.at[didx], gd_v, sem)
        cp1.wait()
        cp2.wait()

        def rstep(r, carry2):
            for t in range(64 // 16):
                go_v[r, pl.ds(t * 16, 16)] = (
                    gs_v[r, pl.ds(t * 16, 16)]
                    + gd_v[r, pl.ds(64 + t * 16, 16)])
            return carry2

        lax.fori_loop(0, K3, rstep, 0)
        pltpu.sync_copy(go_v, g_out.at[pl.ds(base, K3)])
        return carry

    lax.fori_loop(0, C3, chunk, 0)


def _sc_gfin(pp, srcp, dstp):
    fn = pl.kernel(
        _sc_gfin_body,
        out_type=jax.ShapeDtypeStruct((EPF, 128), jnp.float32),
        mesh=plsc.VectorSubcoreMesh(**_MESH),
        compiler_params=_SC_PARAMS,
        scratch_types=[
            pltpu.VMEM((K3,), jnp.int32),
            pltpu.VMEM((K3,), jnp.int32),
            pltpu.VMEM((K3, 128), jnp.float32),
            pltpu.VMEM((K3, 128), jnp.float32),
            pltpu.VMEM((K3, 128), jnp.float32),
            pltpu.SemaphoreType.DMA,
        ],
    )
    return fn(pp, srcp, dstp)


def kernel(x, edge_index, e, xbatch, bn_node_g, bn_node_b, bn_edge_g,
           bn_edge_b, lin0_W, lin0_b, attn_beta, lin_W, lin_b,
           em_W1, em_b1, em_W2, em_b2, em_W3, em_b3, em_W4, em_b4,
           em_W5, em_b5):
    f32 = jnp.float32
    src = edge_index[0]
    dst = edge_index[1]
    loops = jnp.arange(N, dtype=jnp.int32)
    padi = jnp.arange(EP - EL, dtype=jnp.int32) % N
    srcp = jnp.concatenate([src, loops, padi])
    dstp = jnp.concatenate([dst, loops, padi])
    zn = jnp.zeros((N,), f32)
    zacc = jnp.zeros((N, HH), f32)

    xstats = _colstats(x, NB)
    estats = _colstats(e, EBLK)

    xlo, xhi, xn = _node0(
        x, xstats, bn_node_g.reshape(1, -1), bn_node_b.reshape(1, -1),
        lin0_W, lin0_b.reshape(1, -1))

    pp = None
    for i in range(NUM_MP):
        bs = jnp.stack([attn_beta[i], jnp.abs(attn_beta[i])]).reshape(2, 1)
        xnp = jnp.pad(xn, ((0, NP2 - N), (0, 0)))
        ext = _gram(xnp, bs).reshape(N * NSLAB, 128)
        acc_lo, acc_hi, den = _sc_round(xlo, xhi, srcp, dstp, ext, zacc, zn)
        dp = den.reshape(N, 1)
        if i < NUM_MP - 1:
            xlo, xhi, xn = _mp_mid(
                acc_lo, acc_hi, dp, lin_W[i], lin_b[i].reshape(1, -1))
        else:
            pp = _mp_last(
                acc_lo, acc_hi, dp, lin_W[i], lin_b[i].reshape(1, -1),
                em_W1[:H], em_W1[H:2 * H])

    G = _sc_gfin(pp, srcp, dstp)
    return _mlp(
        G, e, estats, bn_edge_g.reshape(1, -1), bn_edge_b.reshape(1, -1),
        em_W1[2 * H:], em_b1.reshape(1, -1), em_W2, em_b2.reshape(1, -1),
        em_W3, em_b3.reshape(1, -1), em_W4, em_b4.reshape(1, -1),
        em_W5, em_b5.reshape(1, -1))
